# baseline (speedup 1.0000x reference)
import functools

import jax
import jax.numpy as jnp
from jax import lax
from jax.experimental import pallas as pl
from jax.experimental.pallas import tpu as pltpu
from jax.experimental.pallas import tpu_sc as plsc

B = 1024
L = 50
V = 1000
LANES = 16
NUM_WORKERS = 32
ROWS_PER_W = B // NUM_WORKERS
CHUNK = 8
INV_L = 1.0 / L

_mesh = plsc.VectorSubcoreMesh(core_axis_name="c", subcore_axis_name="s")


@functools.partial(
    pl.kernel,
    out_type=jax.ShapeDtypeStruct((B, V), jnp.float32),
    mesh=_mesh,
    scratch_types=[
        pltpu.VMEM((ROWS_PER_W, L), jnp.int32),
        pltpu.VMEM((ROWS_PER_W, V), jnp.float32),
        pltpu.SemaphoreType.DMA,
    ],
    compiler_params=pltpu.CompilerParams(needs_layout_passes=False),
)
def _histogram_kernel(assign_hbm, out_hbm, idx_v, cnt_v, sem):
    wid = lax.axis_index("c") * 16 + lax.axis_index("s")
    base = wid * ROWS_PER_W

    pltpu.sync_copy(assign_hbm.at[pl.ds(base, ROWS_PER_W)], idx_v)

    zeros = jnp.zeros((LANES,), jnp.float32)
    val = jnp.full((LANES,), INV_L, jnp.float32)
    lane = lax.iota(jnp.int32, LANES)
    tail_mask = lane >= (LANES - (L - 3 * LANES))
    full_mask = lane >= 0

    for c in range(ROWS_PER_W // CHUNK):
        for r in range(c * CHUNK, (c + 1) * CHUNK):
            for j in range(V // LANES + 1):
                off = min(j * LANES, V - LANES)
                cnt_v[r, pl.ds(off, LANES)] = zeros
            rowv = jnp.full((LANES,), r, jnp.int32)
            for off, mask in ((0, full_mask), (16, full_mask),
                              (32, full_mask), (L - LANES, tail_mask)):
                col = idx_v[r, pl.ds(off, LANES)]
                plsc.addupdate_scatter(cnt_v, [rowv, col], val, mask=mask)
        pltpu.async_copy(cnt_v.at[pl.ds(c * CHUNK, CHUNK)],
                         out_hbm.at[pl.ds(base + c * CHUNK, CHUNK)], sem)

    for c in range(ROWS_PER_W // CHUNK):
        pltpu.make_async_copy(cnt_v.at[pl.ds(c * CHUNK, CHUNK)],
                              out_hbm.at[pl.ds(base + c * CHUNK, CHUNK)],
                              sem).wait()


def kernel(assignments):
    return _histogram_kernel(assignments)


# speedup vs baseline: 1.0024x; 1.0024x over previous
import functools

import jax
import jax.numpy as jnp
from jax import lax
from jax.experimental import pallas as pl
from jax.experimental.pallas import tpu as pltpu
from jax.experimental.pallas import tpu_sc as plsc

B = 1024
L = 50
V = 1000
LANES = 16
NUM_WORKERS = 32
HALF = B // 2
ROWS_PER_W = HALF // NUM_WORKERS
CHUNK = 8
INV_L = 1.0 / L

_mesh = plsc.VectorSubcoreMesh(core_axis_name="c", subcore_axis_name="s")


@functools.partial(
    pl.kernel,
    out_type=jax.ShapeDtypeStruct((HALF, V), jnp.float32),
    mesh=_mesh,
    scratch_types=[
        pltpu.VMEM((ROWS_PER_W, L), jnp.int32),
        pltpu.VMEM((ROWS_PER_W, V), jnp.float32),
        pltpu.SemaphoreType.DMA,
    ],
    compiler_params=pltpu.CompilerParams(needs_layout_passes=False),
)
def _histogram_kernel(assign_hbm, out_hbm, idx_v, cnt_v, sem):
    wid = lax.axis_index("c") * 16 + lax.axis_index("s")
    base = wid * ROWS_PER_W

    pltpu.sync_copy(assign_hbm.at[pl.ds(base, ROWS_PER_W)], idx_v)

    zeros = jnp.zeros((LANES,), jnp.float32)
    val = jnp.full((LANES,), INV_L, jnp.float32)
    lane = lax.iota(jnp.int32, LANES)
    tail_mask = lane >= (LANES - (L - 3 * LANES))
    full_mask = lane >= 0

    def row_body(r, carry):
        for j in range(V // LANES + 1):
            off = min(j * LANES, V - LANES)
            cnt_v[r, pl.ds(off, LANES)] = zeros
        rowv = jnp.full((LANES,), r, jnp.int32)
        for off, mask in ((0, full_mask), (16, full_mask),
                          (32, full_mask), (L - LANES, tail_mask)):
            col = idx_v[r, pl.ds(off, LANES)]
            plsc.addupdate_scatter(cnt_v, [rowv, col], val, mask=mask)
        return carry

    for c in range(ROWS_PER_W // CHUNK):
        lax.fori_loop(c * CHUNK, (c + 1) * CHUNK, row_body, None)
        pltpu.async_copy(cnt_v.at[pl.ds(c * CHUNK, CHUNK)],
                         out_hbm.at[pl.ds(base + c * CHUNK, CHUNK)], sem)

    for c in range(ROWS_PER_W // CHUNK):
        pltpu.make_async_copy(cnt_v.at[pl.ds(c * CHUNK, CHUNK)],
                              out_hbm.at[pl.ds(base + c * CHUNK, CHUNK)],
                              sem).wait()


def kernel(assignments):
    top = _histogram_kernel(lax.slice(assignments, (0, 0), (HALF, L)))
    bot = _histogram_kernel(lax.slice(assignments, (HALF, 0), (B, L)))
    return jnp.concatenate([top, bot], axis=0)


# speedup vs baseline: 1.2176x; 1.2147x over previous
import functools

import jax
import jax.numpy as jnp
from jax import lax
from jax.experimental import pallas as pl
from jax.experimental.pallas import tpu as pltpu
from jax.experimental.pallas import tpu_sc as plsc

B = 1024
L = 50
V = 1000
LANES = 16
NUM_WORKERS = 32
ROWS_PER_W = B // NUM_WORKERS
CHUNK = 8
INV_L = 1.0 / L

_mesh = plsc.VectorSubcoreMesh(core_axis_name="c", subcore_axis_name="s")


@functools.partial(
    pl.kernel,
    out_type=jax.ShapeDtypeStruct((B, V), jnp.float32),
    mesh=_mesh,
    scratch_types=[
        pltpu.VMEM((ROWS_PER_W, L), jnp.int32),
        pltpu.VMEM((ROWS_PER_W, V), jnp.float32),
        pltpu.SemaphoreType.DMA,
        pltpu.SemaphoreType.DMA,
    ],
    compiler_params=pltpu.CompilerParams(needs_layout_passes=False),
)
def _histogram_kernel(assign_hbm, out_hbm, idx_v, cnt_v, sem_in, sem_out):
    wid = lax.axis_index("c") * 16 + lax.axis_index("s")
    base = wid * ROWS_PER_W

    pltpu.async_copy(assign_hbm.at[pl.ds(base, ROWS_PER_W)], idx_v, sem_in)

    zeros = jnp.zeros((LANES,), jnp.float32)
    val = jnp.full((LANES,), INV_L, jnp.float32)
    lane = lax.iota(jnp.int32, LANES)
    tail_mask = lane >= (LANES - (L - 3 * LANES))
    full_mask = lane >= 0

    def zero_body(r, carry):
        for j in range(V // LANES + 1):
            off = min(j * LANES, V - LANES)
            cnt_v[r, pl.ds(off, LANES)] = zeros
        return carry

    lax.fori_loop(0, ROWS_PER_W, zero_body, None)

    pltpu.make_async_copy(assign_hbm.at[pl.ds(base, ROWS_PER_W)], idx_v,
                          sem_in).wait()

    def scatter_body(r, carry):
        rowv = jnp.full((LANES,), r, jnp.int32)
        for off, mask in ((0, full_mask), (16, full_mask),
                          (32, full_mask), (L - LANES, tail_mask)):
            col = idx_v[r, pl.ds(off, LANES)]
            plsc.addupdate_scatter(cnt_v, [rowv, col], val, mask=mask)
        return carry

    for c in range(ROWS_PER_W // CHUNK):
        lax.fori_loop(c * CHUNK, (c + 1) * CHUNK, scatter_body, None)
        pltpu.async_copy(cnt_v.at[pl.ds(c * CHUNK, CHUNK)],
                         out_hbm.at[pl.ds(base + c * CHUNK, CHUNK)], sem_out)

    for c in range(ROWS_PER_W // CHUNK):
        pltpu.make_async_copy(cnt_v.at[pl.ds(c * CHUNK, CHUNK)],
                              out_hbm.at[pl.ds(base + c * CHUNK, CHUNK)],
                              sem_out).wait()


def kernel(assignments):
    return _histogram_kernel(assignments)


# speedup vs baseline: 1.2216x; 1.0033x over previous
import functools

import jax
import jax.numpy as jnp
from jax import lax
from jax.experimental import pallas as pl
from jax.experimental.pallas import tpu as pltpu
from jax.experimental.pallas import tpu_sc as plsc

B = 1024
L = 50
V = 1000
LANES = 16
NUM_WORKERS = 32
ROWS_PER_W = B // NUM_WORKERS
CHUNK = 16
INV_L = 1.0 / L

_mesh = plsc.VectorSubcoreMesh(core_axis_name="c", subcore_axis_name="s")


@functools.partial(
    pl.kernel,
    out_type=jax.ShapeDtypeStruct((B, V), jnp.float32),
    mesh=_mesh,
    scratch_types=[
        pltpu.VMEM((ROWS_PER_W, L), jnp.int32),
        pltpu.VMEM((ROWS_PER_W, V), jnp.float32),
        pltpu.SemaphoreType.DMA,
        pltpu.SemaphoreType.DMA,
    ],
    compiler_params=pltpu.CompilerParams(needs_layout_passes=False),
)
def _histogram_kernel(assign_hbm, out_hbm, idx_v, cnt_v, sem_in, sem_out):
    wid = lax.axis_index("c") * 16 + lax.axis_index("s")
    base = wid * ROWS_PER_W

    pltpu.async_copy(assign_hbm.at[pl.ds(base, ROWS_PER_W)], idx_v, sem_in)

    zeros = jnp.zeros((LANES,), jnp.float32)
    val = jnp.full((LANES,), INV_L, jnp.float32)
    lane = lax.iota(jnp.int32, LANES)
    tail_mask = lane >= (LANES - (L - 3 * LANES))
    full_mask = lane >= 0

    def zero_body(r, carry):
        for j in range(V // LANES + 1):
            off = min(j * LANES, V - LANES)
            cnt_v[r, pl.ds(off, LANES)] = zeros
        return carry

    lax.fori_loop(0, ROWS_PER_W, zero_body, None)

    pltpu.make_async_copy(assign_hbm.at[pl.ds(base, ROWS_PER_W)], idx_v,
                          sem_in).wait()

    def scatter_body(r, carry):
        rowv = jnp.full((LANES,), r, jnp.int32)
        for off, mask in ((0, full_mask), (16, full_mask),
                          (32, full_mask), (L - LANES, tail_mask)):
            col = idx_v[r, pl.ds(off, LANES)]
            plsc.addupdate_scatter(cnt_v, [rowv, col], val, mask=mask)
        return carry

    for c in range(ROWS_PER_W // CHUNK):
        lax.fori_loop(c * CHUNK, (c + 1) * CHUNK, scatter_body, None)
        pltpu.async_copy(cnt_v.at[pl.ds(c * CHUNK, CHUNK)],
                         out_hbm.at[pl.ds(base + c * CHUNK, CHUNK)], sem_out)

    for c in range(ROWS_PER_W // CHUNK):
        pltpu.make_async_copy(cnt_v.at[pl.ds(c * CHUNK, CHUNK)],
                              out_hbm.at[pl.ds(base + c * CHUNK, CHUNK)],
                              sem_out).wait()


def kernel(assignments):
    return _histogram_kernel(assignments)
